# Initial kernel scaffold; baseline (speedup 1.0000x reference)
import jax
import jax.numpy as jnp
from jax.experimental import pallas as pl


def kernel(x, edge_index, edge_attr, lin_W, lin_b, msg_W1, msg_b1, msg_W2, msg_b2, upd_W1, upd_b1, upd_W2, upd_b2):
    raise NotImplementedError("write your pallas kernel here")


# speedup vs baseline: 1.6920x; 1.6920x over previous
import functools

import jax
import jax.numpy as jnp
from jax import lax
from jax.experimental import pallas as pl
from jax.experimental.pallas import tpu as pltpu
from jax.experimental.pallas import tpu_sc as plsc

_N = 10000
_E = 320000
_D = 128
_DE = 16
_L = 4

_NC = 2
_NS = 16
_LANES = 16
_NW = _NC * _NS

_BATCH = 128
_BPW = 80
_EPAD = _NW * _BPW * _BATCH
_AGG_ROWS = 10240
_ZB = _AGG_ROWS // _NS // _BATCH
_ROWS_OUT = _AGG_ROWS // _NS

_EBLK = 4096
_NBLK = 1000


_MESH = plsc.VectorSubcoreMesh(core_axis_name="c", subcore_axis_name="s")


def _sc_gather_body(a_hbm, b_hbm, dstg_hbm, srcg_hbm, out_hbm,
                    idx_d, idx_s, rows_a, rows_b, sem):
    c = lax.axis_index("c")
    s = lax.axis_index("s")
    wid = s * _NC + c
    base = wid * _BPW
    pltpu.sync_copy(dstg_hbm.at[pl.ds(base, _BPW)], idx_d)
    pltpu.sync_copy(srcg_hbm.at[pl.ds(base, _BPW)], idx_s)

    def body(t, carry):
        pltpu.async_copy(a_hbm.at[idx_d.at[t]], rows_a, sem).wait()
        pltpu.async_copy(b_hbm.at[idx_s.at[t]], rows_b, sem).wait()

        def add_row(i, carry2):
            for k in range(_D // _LANES):
                sl = pl.ds(k * _LANES, _LANES)
                rows_a[i, sl] = rows_a[i, sl] + rows_b[i, sl]
            return carry2

        lax.fori_loop(0, _BATCH, add_row, 0)
        pltpu.sync_copy(rows_a, out_hbm.at[pl.ds((base + t) * _BATCH, _BATCH)])
        return carry

    lax.fori_loop(0, _BPW, body, 0)


@functools.partial(
    pl.kernel,
    out_type=jax.ShapeDtypeStruct((_EPAD, _D), jnp.float32),
    mesh=_MESH,
    scratch_types=[
        pltpu.VMEM((_BPW, _BATCH), jnp.int32),
        pltpu.VMEM((_BPW, _BATCH), jnp.int32),
        pltpu.VMEM((_BATCH, _D), jnp.float32),
        pltpu.VMEM((_BATCH, _D), jnp.float32),
        pltpu.SemaphoreType.DMA,
    ],
)
def _sc_gather(a_hbm, b_hbm, dstg_hbm, srcg_hbm, out_hbm,
               idx_d, idx_s, rows_a, rows_b, sem):
    _sc_gather_body(a_hbm, b_hbm, dstg_hbm, srcg_hbm, out_hbm,
                    idx_d, idx_s, rows_a, rows_b, sem)


def _sc_scatter_body(m2_hbm, dsts_hbm, out_hbm, idx_d, rows, aggr, sem):
    c = lax.axis_index("c")
    s = lax.axis_index("s")
    wid = s * _NC + c
    base = wid * _BPW

    def zero_row(i, carry):
        for k in range(_D // _LANES):
            rows[i, pl.ds(k * _LANES, _LANES)] = jnp.zeros((_LANES,), jnp.float32)
        return carry

    lax.fori_loop(0, _BATCH, zero_row, 0)
    for r in range(_ZB):
        pltpu.sync_copy(rows, aggr.at[pl.ds(s * _ROWS_OUT + r * _BATCH, _BATCH)])
    plsc.subcore_barrier()

    pltpu.sync_copy(dsts_hbm.at[pl.ds(base, _BPW)], idx_d)

    def body(t, carry):
        pltpu.sync_copy(m2_hbm.at[pl.ds((base + t) * _BATCH, _BATCH)], rows)
        pltpu.sync_copy(rows, aggr.at[idx_d.at[t]], add=True)
        return carry

    lax.fori_loop(0, _BPW, body, 0)
    plsc.subcore_barrier()
    pltpu.sync_copy(aggr.at[pl.ds(s * _ROWS_OUT, _ROWS_OUT)],
                    out_hbm.at[c, pl.ds(s * _ROWS_OUT, _ROWS_OUT)])


@functools.partial(
    pl.kernel,
    out_type=jax.ShapeDtypeStruct((_NC, _AGG_ROWS, _D), jnp.float32),
    mesh=_MESH,
    scratch_types=[
        pltpu.VMEM((_BPW, _BATCH), jnp.int32),
        pltpu.VMEM((_BATCH, _D), jnp.float32),
        pltpu.VMEM_SHARED((_AGG_ROWS, _D), jnp.float32),
        pltpu.SemaphoreType.DMA,
    ],
)
def _sc_scatter(m2_hbm, dsts_hbm, out_hbm, idx_d, rows, aggr, sem):
    _sc_scatter_body(m2_hbm, dsts_hbm, out_hbm, idx_d, rows, aggr, sem)


def _prep_body(x_ref, w_ref, b_ref, w1i_ref, w1j_ref, h_ref, a_ref, bb_ref):
    h = jnp.dot(x_ref[...], w_ref[...], preferred_element_type=jnp.float32)
    h = h + b_ref[...]
    h_ref[...] = h
    a_ref[...] = jnp.dot(h, w1i_ref[...], preferred_element_type=jnp.float32)
    bb_ref[...] = jnp.dot(h, w1j_ref[...], preferred_element_type=jnp.float32)


def _edge_body(g_ref, ea_ref, w1e_ref, b1_ref, w2_ref, b2_ref, out_ref):
    pre = g_ref[...] + jnp.dot(ea_ref[...], w1e_ref[...],
                               preferred_element_type=jnp.float32) + b1_ref[...]
    m = jnp.tanh(pre)
    out_ref[...] = jnp.tanh(
        jnp.dot(m, w2_ref[...], preferred_element_type=jnp.float32) + b2_ref[...])


def _upd_body(h_ref, p0_ref, p1_ref, wh_ref, wa_ref, b1_ref, w2_ref, b2_ref,
              w1i_ref, w1j_ref, hn_ref, a_ref, bb_ref):
    aggr = p0_ref[...] + p1_ref[...]
    u = jnp.tanh(jnp.dot(h_ref[...], wh_ref[...], preferred_element_type=jnp.float32)
                 + jnp.dot(aggr, wa_ref[...], preferred_element_type=jnp.float32)
                 + b1_ref[...])
    u = jnp.tanh(jnp.dot(u, w2_ref[...], preferred_element_type=jnp.float32)
                 + b2_ref[...])
    hn = h_ref[...] + u
    hn_ref[...] = hn
    a_ref[...] = jnp.dot(hn, w1i_ref[...], preferred_element_type=jnp.float32)
    bb_ref[...] = jnp.dot(hn, w1j_ref[...], preferred_element_type=jnp.float32)


def _full(shape):
    return pl.BlockSpec(shape, lambda i: (0,) * len(shape))


def _rows(blk):
    return pl.BlockSpec((blk, _D), lambda i: (i, 0))


_nds = jax.ShapeDtypeStruct((_N, _D), jnp.float32)

_prep = pl.pallas_call(
    _prep_body,
    grid=(_N // _NBLK,),
    in_specs=[_rows(_NBLK), _full((_D, _D)), _full((1, _D)),
              _full((_D, _D)), _full((_D, _D))],
    out_specs=[_rows(_NBLK)] * 3,
    out_shape=[_nds] * 3,
)

_edge_mlp = pl.pallas_call(
    _edge_body,
    grid=(_EPAD // _EBLK,),
    in_specs=[_rows(_EBLK), pl.BlockSpec((_EBLK, _DE), lambda i: (i, 0)),
              _full((_DE, _D)), _full((1, _D)), _full((_D, _D)), _full((1, _D))],
    out_specs=_rows(_EBLK),
    out_shape=jax.ShapeDtypeStruct((_EPAD, _D), jnp.float32),
)

_update = pl.pallas_call(
    _upd_body,
    grid=(_N // _NBLK,),
    in_specs=[_rows(_NBLK)] * 3 + [_full((_D, _D)), _full((_D, _D)),
              _full((1, _D)), _full((_D, _D)), _full((1, _D)),
              _full((_D, _D)), _full((_D, _D))],
    out_specs=[_rows(_NBLK)] * 3,
    out_shape=[_nds] * 3,
)


def kernel(x, edge_index, edge_attr, lin_W, lin_b,
           msg_W1, msg_b1, msg_W2, msg_b2,
           upd_W1, upd_b1, upd_W2, upd_b2):
    src = edge_index[0].astype(jnp.int32)
    dst = edge_index[1].astype(jnp.int32)
    pad = _EPAD - _E
    dst_g = jnp.concatenate([dst, jnp.zeros((pad,), jnp.int32)]).reshape(-1, _BATCH)
    src_g = jnp.concatenate([src, jnp.zeros((pad,), jnp.int32)]).reshape(-1, _BATCH)
    dst_s = jnp.concatenate([dst, jnp.full((pad,), _N, jnp.int32)]).reshape(-1, _BATCH)
    ea_pad = jnp.concatenate(
        [edge_attr, jnp.zeros((pad, _DE), jnp.float32)], axis=0)

    h, a, b = _prep(x, lin_W, lin_b.reshape(1, _D),
                    msg_W1[0, :_D], msg_W1[0, _D:2 * _D])
    for l in range(_L):
        g = _sc_gather(a, b, dst_g, src_g)
        m2 = _edge_mlp(g, ea_pad, msg_W1[l, 2 * _D:],
                       msg_b1[l].reshape(1, _D), msg_W2[l],
                       msg_b2[l].reshape(1, _D))
        parts = _sc_scatter(m2, dst_s)
        ln = min(l + 1, _L - 1)
        h, a, b = _update(h, parts[0, :_N], parts[1, :_N],
                          upd_W1[l, :_D], upd_W1[l, _D:],
                          upd_b1[l].reshape(1, _D), upd_W2[l],
                          upd_b2[l].reshape(1, _D),
                          msg_W1[ln, :_D], msg_W1[ln, _D:2 * _D])
    return h


# speedup vs baseline: 2.6233x; 1.5504x over previous
import functools

import jax
import jax.numpy as jnp
from jax import lax
from jax.experimental import pallas as pl
from jax.experimental.pallas import tpu as pltpu
from jax.experimental.pallas import tpu_sc as plsc

_N = 10000
_E = 320000
_D = 128
_DE = 16
_L = 4

_NC = 2
_NS = 16
_LANES = 16
_NW = _NC * _NS

_BATCH = 128
_BPW = 80
_EPAD = _NW * _BPW * _BATCH
_AGG_ROWS = 10240
_ZB = _AGG_ROWS // _NS // _BATCH
_ROWS_OUT = _AGG_ROWS // _NS

_EBLK = 4096
_NBLK = 1000


_MESH = plsc.VectorSubcoreMesh(core_axis_name="c", subcore_axis_name="s")


def _sc_gather_body(a_hbm, b_hbm, dstg_hbm, srcg_hbm, out_hbm,
                    idx_d, idx_s, rows_a, rows_b, rows_o,
                    sga, sgb, sout):
    c = lax.axis_index("c")
    s = lax.axis_index("s")
    wid = s * _NC + c
    base = wid * _BPW
    pltpu.sync_copy(dstg_hbm.at[pl.ds(base, _BPW)], idx_d)
    pltpu.sync_copy(srcg_hbm.at[pl.ds(base, _BPW)], idx_s)

    def issue(t, buf):
        pltpu.async_copy(a_hbm.at[idx_d.at[t]], rows_a.at[buf], sga.at[buf])
        pltpu.async_copy(b_hbm.at[idx_s.at[t]], rows_b.at[buf], sgb.at[buf])

    def wait_gather(buf):
        pltpu.make_async_copy(a_hbm.at[idx_d.at[0]], rows_a.at[buf],
                              sga.at[buf]).wait()
        pltpu.make_async_copy(b_hbm.at[idx_s.at[0]], rows_b.at[buf],
                              sgb.at[buf]).wait()

    def add_and_write(t, buf):
        def add_row(i, carry2):
            for k in range(_D // _LANES):
                sl = pl.ds(k * _LANES, _LANES)
                rows_o[buf, i, sl] = rows_a[buf, i, sl] + rows_b[buf, i, sl]
            return carry2

        lax.fori_loop(0, _BATCH, add_row, 0)
        pltpu.async_copy(rows_o.at[buf],
                         out_hbm.at[pl.ds((base + t) * _BATCH, _BATCH)],
                         sout.at[buf])

    def wait_write(buf):
        pltpu.make_async_copy(rows_o.at[buf],
                              out_hbm.at[pl.ds(0, _BATCH)], sout.at[buf]).wait()

    issue(0, 0)

    def body(tt, carry):
        t0 = 2 * tt
        wait_gather(0)
        issue(t0 + 1, 1)

        @pl.when(tt > 0)
        def _():
            wait_write(0)

        add_and_write(t0, 0)
        wait_gather(1)

        @pl.when(tt < _BPW // 2 - 1)
        def _():
            issue(t0 + 2, 0)

        @pl.when(tt > 0)
        def _():
            wait_write(1)

        add_and_write(t0 + 1, 1)
        return carry

    lax.fori_loop(0, _BPW // 2, body, 0)
    wait_write(0)
    wait_write(1)


@functools.partial(
    pl.kernel,
    out_type=jax.ShapeDtypeStruct((_EPAD, _D), jnp.float32),
    mesh=_MESH,
    scratch_types=[
        pltpu.VMEM((_BPW, _BATCH), jnp.int32),
        pltpu.VMEM((_BPW, _BATCH), jnp.int32),
        pltpu.VMEM((2, _BATCH, _D), jnp.float32),
        pltpu.VMEM((2, _BATCH, _D), jnp.float32),
        pltpu.VMEM((2, _BATCH, _D), jnp.float32),
        pltpu.SemaphoreType.DMA((2,)),
        pltpu.SemaphoreType.DMA((2,)),
        pltpu.SemaphoreType.DMA((2,)),
    ],
)
def _sc_gather(a_hbm, b_hbm, dstg_hbm, srcg_hbm, out_hbm,
               idx_d, idx_s, rows_a, rows_b, rows_o, sga, sgb, sout):
    _sc_gather_body(a_hbm, b_hbm, dstg_hbm, srcg_hbm, out_hbm,
                    idx_d, idx_s, rows_a, rows_b, rows_o, sga, sgb, sout)


def _sc_scatter_body(m2_hbm, dsts_hbm, out_hbm, idx_d, rows, aggr, sld, ssc):
    c = lax.axis_index("c")
    s = lax.axis_index("s")
    wid = s * _NC + c
    base = wid * _BPW

    def zero_row(i, carry):
        for k in range(_D // _LANES):
            rows[0, i, pl.ds(k * _LANES, _LANES)] = jnp.zeros((_LANES,),
                                                              jnp.float32)
        return carry

    lax.fori_loop(0, _BATCH, zero_row, 0)
    for r in range(_ZB):
        pltpu.sync_copy(rows.at[0],
                        aggr.at[pl.ds(s * _ROWS_OUT + r * _BATCH, _BATCH)])
    plsc.subcore_barrier()

    pltpu.sync_copy(dsts_hbm.at[pl.ds(base, _BPW)], idx_d)

    def issue_load(t, buf):
        pltpu.async_copy(m2_hbm.at[pl.ds((base + t) * _BATCH, _BATCH)],
                         rows.at[buf], sld.at[buf])

    def wait_load(buf):
        pltpu.make_async_copy(m2_hbm.at[pl.ds(0, _BATCH)], rows.at[buf],
                              sld.at[buf]).wait()

    def issue_scatter(t, buf):
        pltpu.async_copy(rows.at[buf], aggr.at[idx_d.at[t]], ssc.at[buf],
                         add=True)

    def wait_scatter(buf):
        pltpu.make_async_copy(rows.at[buf], aggr.at[idx_d.at[0]],
                              ssc.at[buf]).wait()

    issue_load(0, 0)

    def body(tt, carry):
        t0 = 2 * tt
        wait_load(0)
        issue_load(t0 + 1, 1)

        @pl.when(tt > 0)
        def _():
            wait_scatter(0)

        issue_scatter(t0, 0)
        wait_load(1)

        @pl.when(tt < _BPW // 2 - 1)
        def _():
            issue_load(t0 + 2, 0)

        @pl.when(tt > 0)
        def _():
            wait_scatter(1)

        issue_scatter(t0 + 1, 1)
        return carry

    lax.fori_loop(0, _BPW // 2, body, 0)
    wait_scatter(0)
    wait_scatter(1)
    plsc.subcore_barrier()
    pltpu.sync_copy(aggr.at[pl.ds(s * _ROWS_OUT, _ROWS_OUT)],
                    out_hbm.at[c, pl.ds(s * _ROWS_OUT, _ROWS_OUT)])


@functools.partial(
    pl.kernel,
    out_type=jax.ShapeDtypeStruct((_NC, _AGG_ROWS, _D), jnp.float32),
    mesh=_MESH,
    scratch_types=[
        pltpu.VMEM((_BPW, _BATCH), jnp.int32),
        pltpu.VMEM((2, _BATCH, _D), jnp.float32),
        pltpu.VMEM_SHARED((_AGG_ROWS, _D), jnp.float32),
        pltpu.SemaphoreType.DMA((2,)),
        pltpu.SemaphoreType.DMA((2,)),
    ],
)
def _sc_scatter(m2_hbm, dsts_hbm, out_hbm, idx_d, rows, aggr, sld, ssc):
    _sc_scatter_body(m2_hbm, dsts_hbm, out_hbm, idx_d, rows, aggr, sld, ssc)


def _prep_body(x_ref, w_ref, b_ref, w1i_ref, w1j_ref, h_ref, a_ref, bb_ref):
    h = jnp.dot(x_ref[...], w_ref[...], preferred_element_type=jnp.float32)
    h = h + b_ref[...]
    h_ref[...] = h
    a_ref[...] = jnp.dot(h, w1i_ref[...], preferred_element_type=jnp.float32)
    bb_ref[...] = jnp.dot(h, w1j_ref[...], preferred_element_type=jnp.float32)


def _edge_body(g_ref, ea_ref, w1e_ref, b1_ref, w2_ref, b2_ref, out_ref):
    pre = g_ref[...] + jnp.dot(ea_ref[...], w1e_ref[...],
                               preferred_element_type=jnp.float32) + b1_ref[...]
    m = jnp.tanh(pre)
    out_ref[...] = jnp.tanh(
        jnp.dot(m, w2_ref[...], preferred_element_type=jnp.float32) + b2_ref[...])


def _upd_body(h_ref, p0_ref, p1_ref, wh_ref, wa_ref, b1_ref, w2_ref, b2_ref,
              w1i_ref, w1j_ref, hn_ref, a_ref, bb_ref):
    aggr = p0_ref[...] + p1_ref[...]
    u = jnp.tanh(jnp.dot(h_ref[...], wh_ref[...], preferred_element_type=jnp.float32)
                 + jnp.dot(aggr, wa_ref[...], preferred_element_type=jnp.float32)
                 + b1_ref[...])
    u = jnp.tanh(jnp.dot(u, w2_ref[...], preferred_element_type=jnp.float32)
                 + b2_ref[...])
    hn = h_ref[...] + u
    hn_ref[...] = hn
    a_ref[...] = jnp.dot(hn, w1i_ref[...], preferred_element_type=jnp.float32)
    bb_ref[...] = jnp.dot(hn, w1j_ref[...], preferred_element_type=jnp.float32)


def _full(shape):
    return pl.BlockSpec(shape, lambda i: (0,) * len(shape))


def _rows(blk):
    return pl.BlockSpec((blk, _D), lambda i: (i, 0))


_nds = jax.ShapeDtypeStruct((_N, _D), jnp.float32)

_prep = pl.pallas_call(
    _prep_body,
    grid=(_N // _NBLK,),
    in_specs=[_rows(_NBLK), _full((_D, _D)), _full((1, _D)),
              _full((_D, _D)), _full((_D, _D))],
    out_specs=[_rows(_NBLK)] * 3,
    out_shape=[_nds] * 3,
)

_edge_mlp = pl.pallas_call(
    _edge_body,
    grid=(_EPAD // _EBLK,),
    in_specs=[_rows(_EBLK), pl.BlockSpec((_EBLK, _DE), lambda i: (i, 0)),
              _full((_DE, _D)), _full((1, _D)), _full((_D, _D)), _full((1, _D))],
    out_specs=_rows(_EBLK),
    out_shape=jax.ShapeDtypeStruct((_EPAD, _D), jnp.float32),
)

_update = pl.pallas_call(
    _upd_body,
    grid=(_N // _NBLK,),
    in_specs=[_rows(_NBLK)] * 3 + [_full((_D, _D)), _full((_D, _D)),
              _full((1, _D)), _full((_D, _D)), _full((1, _D)),
              _full((_D, _D)), _full((_D, _D))],
    out_specs=[_rows(_NBLK)] * 3,
    out_shape=[_nds] * 3,
)


def kernel(x, edge_index, edge_attr, lin_W, lin_b,
           msg_W1, msg_b1, msg_W2, msg_b2,
           upd_W1, upd_b1, upd_W2, upd_b2):
    src = edge_index[0].astype(jnp.int32)
    dst = edge_index[1].astype(jnp.int32)
    pad = _EPAD - _E
    dst_g = jnp.concatenate([dst, jnp.zeros((pad,), jnp.int32)]).reshape(-1, _BATCH)
    src_g = jnp.concatenate([src, jnp.zeros((pad,), jnp.int32)]).reshape(-1, _BATCH)
    dst_s = jnp.concatenate([dst, jnp.full((pad,), _N, jnp.int32)]).reshape(-1, _BATCH)
    ea_pad = jnp.concatenate(
        [edge_attr, jnp.zeros((pad, _DE), jnp.float32)], axis=0)

    h, a, b = _prep(x, lin_W, lin_b.reshape(1, _D),
                    msg_W1[0, :_D], msg_W1[0, _D:2 * _D])
    for l in range(_L):
        g = _sc_gather(a, b, dst_g, src_g)
        m2 = _edge_mlp(g, ea_pad, msg_W1[l, 2 * _D:],
                       msg_b1[l].reshape(1, _D), msg_W2[l],
                       msg_b2[l].reshape(1, _D))
        parts = _sc_scatter(m2, dst_s)
        ln = min(l + 1, _L - 1)
        h, a, b = _update(h, parts[0, :_N], parts[1, :_N],
                          upd_W1[l, :_D], upd_W1[l, _D:],
                          upd_b1[l].reshape(1, _D), upd_W2[l],
                          upd_b2[l].reshape(1, _D),
                          msg_W1[ln, :_D], msg_W1[ln, _D:2 * _D])
    return h


# speedup vs baseline: 3.0679x; 1.1695x over previous
import functools

import jax
import jax.numpy as jnp
from jax import lax
from jax.experimental import pallas as pl
from jax.experimental.pallas import tpu as pltpu
from jax.experimental.pallas import tpu_sc as plsc

_N = 10000
_E = 320000
_D = 128
_DE = 16
_L = 4

_NC = 2
_NS = 16
_LANES = 16
_NW = _NC * _NS

_BATCH = 128
_BPW = 80
_EPAD = _NW * _BPW * _BATCH
_AGG_ROWS = 10240
_ZB = _AGG_ROWS // _NS // _BATCH
_ROWS_OUT = _AGG_ROWS // _NS

_EBLK = 4096
_NBLK = 1024
_NPAD = 10240


_MESH = plsc.VectorSubcoreMesh(core_axis_name="c", subcore_axis_name="s")


def _make_sc_gather(bpw):
    epad = _NW * bpw * _BATCH

    def gather_body(a_hbm, b_hbm, dstg_hbm, srcg_hbm, out_hbm,
                    idx_d, idx_s, rows_a, rows_b, sga, sgb, sout):
        c = lax.axis_index("c")
        s = lax.axis_index("s")
        wid = s * _NC + c
        base = wid * bpw
        pltpu.sync_copy(dstg_hbm.at[pl.ds(base, bpw)], idx_d)
        pltpu.sync_copy(srcg_hbm.at[pl.ds(base, bpw)], idx_s)

        def issue(t, buf):
            pltpu.async_copy(a_hbm.at[idx_d.at[t]], rows_a.at[buf], sga.at[buf])
            pltpu.async_copy(b_hbm.at[idx_s.at[t]], rows_b.at[buf], sgb.at[buf])

        def wait_gather(buf):
            pltpu.make_async_copy(a_hbm.at[idx_d.at[0]], rows_a.at[buf],
                                  sga.at[buf]).wait()
            pltpu.make_async_copy(b_hbm.at[idx_s.at[0]], rows_b.at[buf],
                                  sgb.at[buf]).wait()

        def add_and_write(t, buf):
            def add_row(i, carry2):
                for k in range(_D // _LANES):
                    sl = pl.ds(k * _LANES, _LANES)
                    rows_a[buf, i, sl] = rows_a[buf, i, sl] + rows_b[buf, i, sl]
                return carry2

            lax.fori_loop(0, _BATCH, add_row, 0)
            pltpu.async_copy(rows_a.at[buf],
                             out_hbm.at[pl.ds((base + t) * _BATCH, _BATCH)],
                             sout.at[buf])

        def wait_write(buf):
            pltpu.make_async_copy(rows_a.at[buf], out_hbm.at[pl.ds(0, _BATCH)],
                                  sout.at[buf]).wait()

        issue(0, 0)
        issue(1, 1)

        def body(tt, carry):
            t0 = 3 * tt
            for b in range(3):
                t = t0 + b
                pb = (b + 2) % 3
                wait_gather(b)
                add_and_write(t, b)
                if b == 0:

                    @pl.when(tt > 0)
                    def _():
                        wait_write(pb)

                else:
                    wait_write(pb)

                @pl.when(t + 2 < bpw)
                def _():
                    issue(t + 2, pb)

            return carry

        lax.fori_loop(0, bpw // 3, body, 0)
        for t in range(bpw - bpw % 3, bpw):
            b = t % 3
            wait_gather(b)
            add_and_write(t, b)
        ntt = bpw // 3
        issued = [len(range(b, bpw, 3)) for b in range(3)]
        waited = [ntt, ntt, ntt - 1]
        for b in range(3):
            for _ in range(issued[b] - waited[b]):
                wait_write(b)

    @functools.partial(
        pl.kernel,
        out_type=jax.ShapeDtypeStruct((epad, _D), jnp.float32),
        mesh=_MESH,
        scratch_types=[
            pltpu.VMEM((bpw, _BATCH), jnp.int32),
            pltpu.VMEM((bpw, _BATCH), jnp.int32),
            pltpu.VMEM((3, _BATCH, _D), jnp.float32),
            pltpu.VMEM((3, _BATCH, _D), jnp.float32),
            pltpu.SemaphoreType.DMA((3,)),
            pltpu.SemaphoreType.DMA((3,)),
            pltpu.SemaphoreType.DMA((3,)),
        ],
    )
    def gather_kernel(a_hbm, b_hbm, dstg_hbm, srcg_hbm, out_hbm,
                      idx_d, idx_s, rows_a, rows_b, sga, sgb, sout):
        gather_body(a_hbm, b_hbm, dstg_hbm, srcg_hbm, out_hbm,
                    idx_d, idx_s, rows_a, rows_b, sga, sgb, sout)

    return gather_kernel


def _make_sc_scatter(bpw):
    def scatter_body(m2_hbm, dsts_hbm, out_hbm, idx_d, rows, aggr, sld, ssc):
        c = lax.axis_index("c")
        s = lax.axis_index("s")
        wid = s * _NC + c
        base = wid * bpw

        def zero_row(i, carry):
            for k in range(_D // _LANES):
                rows[0, i, pl.ds(k * _LANES, _LANES)] = jnp.zeros(
                    (_LANES,), jnp.float32)
            return carry

        lax.fori_loop(0, _BATCH, zero_row, 0)
        for r in range(_ZB):
            pltpu.sync_copy(rows.at[0],
                            aggr.at[pl.ds(s * _ROWS_OUT + r * _BATCH, _BATCH)])
        plsc.subcore_barrier()

        pltpu.sync_copy(dsts_hbm.at[pl.ds(base, bpw)], idx_d)

        def issue_load(t, buf):
            pltpu.async_copy(m2_hbm.at[pl.ds((base + t) * _BATCH, _BATCH)],
                             rows.at[buf], sld.at[buf])

        def wait_load(buf):
            pltpu.make_async_copy(m2_hbm.at[pl.ds(0, _BATCH)], rows.at[buf],
                                  sld.at[buf]).wait()

        def issue_scatter(t, buf):
            pltpu.async_copy(rows.at[buf], aggr.at[idx_d.at[t]], ssc.at[buf],
                             add=True)

        def wait_scatter(buf):
            pltpu.make_async_copy(rows.at[buf], aggr.at[idx_d.at[0]],
                                  ssc.at[buf]).wait()

        issue_load(0, 0)

        def body(tt, carry):
            t0 = 2 * tt
            wait_load(0)
            issue_load(t0 + 1, 1)

            @pl.when(tt > 0)
            def _():
                wait_scatter(0)

            issue_scatter(t0, 0)
            wait_load(1)

            @pl.when(tt < bpw // 2 - 1)
            def _():
                issue_load(t0 + 2, 0)

            @pl.when(tt > 0)
            def _():
                wait_scatter(1)

            issue_scatter(t0 + 1, 1)
            return carry

        lax.fori_loop(0, bpw // 2, body, 0)
        wait_scatter(0)
        wait_scatter(1)
        plsc.subcore_barrier()
        pltpu.sync_copy(aggr.at[pl.ds(s * _ROWS_OUT, _ROWS_OUT)],
                        out_hbm.at[c, pl.ds(s * _ROWS_OUT, _ROWS_OUT)])

    @functools.partial(
        pl.kernel,
        out_type=jax.ShapeDtypeStruct((_NC, _AGG_ROWS, _D), jnp.float32),
        mesh=_MESH,
        scratch_types=[
            pltpu.VMEM((bpw, _BATCH), jnp.int32),
            pltpu.VMEM((2, _BATCH, _D), jnp.float32),
            pltpu.VMEM_SHARED((_AGG_ROWS, _D), jnp.float32),
            pltpu.SemaphoreType.DMA((2,)),
            pltpu.SemaphoreType.DMA((2,)),
        ],
    )
    def scatter_kernel(m2_hbm, dsts_hbm, out_hbm, idx_d, rows, aggr, sld, ssc):
        scatter_body(m2_hbm, dsts_hbm, out_hbm, idx_d, rows, aggr, sld, ssc)

    return scatter_kernel


_HBPW = _BPW // 2
_EHALF = _EPAD // 2
_sc_gather_half = _make_sc_gather(_HBPW)
_sc_scatter_half = _make_sc_scatter(_HBPW)


def _prep_body(x_ref, w_ref, b_ref, w1i_ref, w1j_ref, h_ref, a_ref, bb_ref):
    h = jnp.dot(x_ref[...], w_ref[...], preferred_element_type=jnp.float32)
    h = h + b_ref[...]
    h_ref[...] = h
    a_ref[...] = jnp.dot(h, w1i_ref[...], preferred_element_type=jnp.float32)
    bb_ref[...] = jnp.dot(h, w1j_ref[...], preferred_element_type=jnp.float32)


def _edge_body(g_ref, ea_ref, w1e_ref, b1_ref, w2_ref, b2_ref, out_ref):
    pre = g_ref[...] + jnp.dot(ea_ref[...], w1e_ref[...],
                               preferred_element_type=jnp.float32) + b1_ref[...]
    m = jnp.tanh(pre)
    out_ref[...] = jnp.tanh(
        jnp.dot(m, w2_ref[...], preferred_element_type=jnp.float32) + b2_ref[...])


def _upd_body(h_ref, p0_ref, p1_ref, p2_ref, p3_ref,
              wh_ref, wa_ref, b1_ref, w2_ref, b2_ref,
              w1i_ref, w1j_ref, hn_ref, a_ref, bb_ref):
    aggr = (p0_ref[...] + p1_ref[...]) + (p2_ref[...] + p3_ref[...])
    u = jnp.tanh(jnp.dot(h_ref[...], wh_ref[...], preferred_element_type=jnp.float32)
                 + jnp.dot(aggr, wa_ref[...], preferred_element_type=jnp.float32)
                 + b1_ref[...])
    u = jnp.tanh(jnp.dot(u, w2_ref[...], preferred_element_type=jnp.float32)
                 + b2_ref[...])
    hn = h_ref[...] + u
    hn_ref[...] = hn
    a_ref[...] = jnp.dot(hn, w1i_ref[...], preferred_element_type=jnp.float32)
    bb_ref[...] = jnp.dot(hn, w1j_ref[...], preferred_element_type=jnp.float32)


def _full(shape):
    return pl.BlockSpec(shape, lambda i: (0,) * len(shape))


def _rows(blk):
    return pl.BlockSpec((blk, _D), lambda i: (i, 0))


_nds = jax.ShapeDtypeStruct((_N, _D), jnp.float32)
_ndsp = jax.ShapeDtypeStruct((_NPAD, _D), jnp.float32)

_prep = pl.pallas_call(
    _prep_body,
    grid=(_NPAD // _NBLK,),
    in_specs=[_rows(_NBLK), _full((_D, _D)), _full((1, _D)),
              _full((_D, _D)), _full((_D, _D))],
    out_specs=[_rows(_NBLK)] * 3,
    out_shape=[_nds, _ndsp, _ndsp],
)

_edge_mlp_half = pl.pallas_call(
    _edge_body,
    grid=(_EHALF // _EBLK,),
    in_specs=[_rows(_EBLK), pl.BlockSpec((_EBLK, _DE), lambda i: (i, 0)),
              _full((_DE, _D)), _full((1, _D)), _full((_D, _D)), _full((1, _D))],
    out_specs=_rows(_EBLK),
    out_shape=jax.ShapeDtypeStruct((_EHALF, _D), jnp.float32),
)

_update = pl.pallas_call(
    _upd_body,
    grid=(_NPAD // _NBLK,),
    in_specs=[_rows(_NBLK)] * 5 + [_full((_D, _D)), _full((_D, _D)),
              _full((1, _D)), _full((_D, _D)), _full((1, _D)),
              _full((_D, _D)), _full((_D, _D))],
    out_specs=[_rows(_NBLK)] * 3,
    out_shape=[_nds, _ndsp, _ndsp],
)


def kernel(x, edge_index, edge_attr, lin_W, lin_b,
           msg_W1, msg_b1, msg_W2, msg_b2,
           upd_W1, upd_b1, upd_W2, upd_b2):
    src = edge_index[0].astype(jnp.int32)
    dst = edge_index[1].astype(jnp.int32)
    pad = _EPAD - _E
    dst_g = jnp.concatenate([dst, jnp.zeros((pad,), jnp.int32)]).reshape(-1, _BATCH)
    src_g = jnp.concatenate([src, jnp.zeros((pad,), jnp.int32)]).reshape(-1, _BATCH)
    dst_s = jnp.concatenate([dst, jnp.full((pad,), _N, jnp.int32)]).reshape(-1, _BATCH)
    ea_pad = jnp.concatenate(
        [edge_attr, jnp.zeros((pad, _DE), jnp.float32)], axis=0)

    nbh = _EHALF // _BATCH
    dg = (dst_g[:nbh], dst_g[nbh:])
    sg = (src_g[:nbh], src_g[nbh:])
    ds2 = (dst_s[:nbh], dst_s[nbh:])
    ea2 = (ea_pad[:_EHALF], ea_pad[_EHALF:])

    h, a, b = _prep(x, lin_W, lin_b.reshape(1, _D),
                    msg_W1[0, :_D], msg_W1[0, _D:2 * _D])
    for l in range(_L):
        w1e = msg_W1[l, 2 * _D:]
        b1 = msg_b1[l].reshape(1, _D)
        w2 = msg_W2[l]
        b2 = msg_b2[l].reshape(1, _D)
        parts = []
        gs = [_sc_gather_half(a, b, dg[q], sg[q]) for q in range(2)]
        ms = [_edge_mlp_half(gs[q], ea2[q], w1e, b1, w2, b2) for q in range(2)]
        parts = [_sc_scatter_half(ms[q], ds2[q]) for q in range(2)]
        ln = min(l + 1, _L - 1)
        h, a, b = _update(h, parts[0][0, :_N], parts[0][1, :_N],
                          parts[1][0, :_N], parts[1][1, :_N],
                          upd_W1[l, :_D], upd_W1[l, _D:],
                          upd_b1[l].reshape(1, _D), upd_W2[l],
                          upd_b2[l].reshape(1, _D),
                          msg_W1[ln, :_D], msg_W1[ln, _D:2 * _D])
    return h
